# baseline (speedup 1.0000x reference)
import functools

import jax
import jax.numpy as jnp
from jax import lax
from jax.experimental import pallas as pl
from jax.experimental.pallas import tpu as pltpu
from jax.experimental.pallas import tpu_sc as plsc

VOCAB = 1_000_000
DIM_E = 64
BATCH = 4096
HIST = 200

_NC = 2
_NS = 16
_NW = _NC * _NS

_B = BATCH * HIST
_PER_W = _B // _NW
_CH = 128
_NCH = _PER_W // _CH


@functools.partial(
    pl.kernel,
    mesh=plsc.VectorSubcoreMesh(core_axis_name="c", subcore_axis_name="s"),
    out_type=jax.ShapeDtypeStruct((_B, DIM_E), jnp.float32),
    scratch_types=[
        pltpu.VMEM((_NCH, _CH), jnp.int32),
        pltpu.VMEM((_CH, DIM_E), jnp.float32),
        pltpu.SemaphoreType.DMA,
    ],
    compiler_params=pltpu.CompilerParams(use_tc_tiling_on_sc=False),
)
def _gather_kernel(idx_hbm, table_hbm, out_hbm, idx_v, rows_v, sem):
    wid = lax.axis_index("s") * _NC + lax.axis_index("c")
    base = wid * _PER_W
    pltpu.sync_copy(idx_hbm.at[wid], idx_v)

    def body(j, carry):
        pltpu.async_copy(table_hbm.at[idx_v.at[j]], rows_v, sem).wait()
        pltpu.sync_copy(rows_v, out_hbm.at[pl.ds(base + j * _CH, _CH)])
        return carry

    lax.fori_loop(0, _NCH, body, 0)


def kernel(y, table):
    idx = y.astype(jnp.int32).reshape(_NW, _NCH, _CH)
    out = _gather_kernel(idx, table)
    return out.reshape(BATCH, HIST, DIM_E)


# speedup vs baseline: 1.1151x; 1.1151x over previous
import functools

import jax
import jax.numpy as jnp
from jax import lax
from jax.experimental import pallas as pl
from jax.experimental.pallas import tpu as pltpu
from jax.experimental.pallas import tpu_sc as plsc

VOCAB = 1_000_000
DIM_E = 64
BATCH = 4096
HIST = 200

_NC = 2
_NS = 16
_NW = _NC * _NS

_B = BATCH * HIST
_PER_W = _B // _NW
_CH = 128
_NCH = _PER_W // _CH
_NBUF = 4
_LA = 2
_NGRP = _NCH // _NBUF


@functools.partial(
    pl.kernel,
    mesh=plsc.VectorSubcoreMesh(core_axis_name="c", subcore_axis_name="s"),
    out_type=jax.ShapeDtypeStruct((_B, DIM_E), jnp.float32),
    scratch_types=[
        pltpu.VMEM((_NCH, _CH), jnp.int32),
        pltpu.VMEM((_NBUF, _CH, DIM_E), jnp.float32),
        pltpu.SemaphoreType.DMA((_NBUF,)),
        pltpu.SemaphoreType.DMA((_NBUF,)),
    ],
    compiler_params=pltpu.CompilerParams(use_tc_tiling_on_sc=False),
)
def _gather_kernel(idx_hbm, table_hbm, out_hbm, idx_v, rows_v, gsem, osem):
    wid = lax.axis_index("s") * _NC + lax.axis_index("c")
    base = wid * _PER_W
    pltpu.sync_copy(idx_hbm.at[wid], idx_v)

    def gather(j, b):
        return pltpu.make_async_copy(
            table_hbm.at[idx_v.at[j]], rows_v.at[b], gsem.at[b])

    def out_copy(j, b):
        return pltpu.make_async_copy(
            rows_v.at[b], out_hbm.at[pl.ds(base + j * _CH, _CH)], osem.at[b])

    for k in range(_NBUF):
        gather(k, k).start()
        jc = k - _LA
        if jc >= 0:
            bc = jc % _NBUF
            gather(jc, bc).wait()
            out_copy(jc, bc).start()

    def body(g, carry):
        for k in range(_NBUF):
            j = g * _NBUF + k
            out_copy(j - _NBUF, k).wait()
            gather(j, k).start()
            jc = j - _LA
            bc = (k - _LA) % _NBUF
            gather(jc, bc).wait()
            out_copy(jc, bc).start()
        return carry

    lax.fori_loop(1, _NGRP, body, 0)

    for jc in range(_NCH - _LA, _NCH):
        bc = jc % _NBUF
        gather(jc, bc).wait()
        out_copy(jc, bc).start()
    for k in range(_NBUF):
        out_copy(_NCH - _NBUF + k, k).wait()


def kernel(y, table):
    idx = y.astype(jnp.int32).reshape(_NW, _NCH, _CH)
    out = _gather_kernel(idx, table)
    return out.reshape(BATCH, HIST, DIM_E)


# speedup vs baseline: 1.1160x; 1.0008x over previous
import functools

import jax
import jax.numpy as jnp
from jax import lax
from jax.experimental import pallas as pl
from jax.experimental.pallas import tpu as pltpu
from jax.experimental.pallas import tpu_sc as plsc

VOCAB = 1_000_000
DIM_E = 64
BATCH = 4096
HIST = 200

_NC = 2
_NS = 16
_NW = _NC * _NS

_B = BATCH * HIST
_PER_W = _B // _NW
_CH = 256
_NCH = _PER_W // _CH
_NBUF = 4
_LA = 2
_NGRP = _NCH // _NBUF


@functools.partial(
    pl.kernel,
    mesh=plsc.VectorSubcoreMesh(core_axis_name="c", subcore_axis_name="s"),
    out_type=jax.ShapeDtypeStruct((_B, DIM_E), jnp.float32),
    scratch_types=[
        pltpu.VMEM((_NCH, _CH), jnp.int32),
        pltpu.VMEM((_NBUF, _CH, DIM_E), jnp.float32),
        pltpu.SemaphoreType.DMA((_NBUF,)),
        pltpu.SemaphoreType.DMA((_NBUF,)),
    ],
    compiler_params=pltpu.CompilerParams(use_tc_tiling_on_sc=False),
)
def _gather_kernel(idx_hbm, table_hbm, out_hbm, idx_v, rows_v, gsem, osem):
    wid = lax.axis_index("s") * _NC + lax.axis_index("c")
    base = wid * _PER_W
    pltpu.sync_copy(idx_hbm.at[wid], idx_v)

    def gather(j, b):
        return pltpu.make_async_copy(
            table_hbm.at[idx_v.at[j]], rows_v.at[b], gsem.at[b])

    def out_copy(j, b):
        return pltpu.make_async_copy(
            rows_v.at[b], out_hbm.at[pl.ds(base + j * _CH, _CH)], osem.at[b])

    for k in range(_NBUF):
        gather(k, k).start()
        jc = k - _LA
        if jc >= 0:
            bc = jc % _NBUF
            gather(jc, bc).wait()
            out_copy(jc, bc).start()

    def body(g, carry):
        for k in range(_NBUF):
            j = g * _NBUF + k
            out_copy(j - _NBUF, k).wait()
            gather(j, k).start()
            jc = j - _LA
            bc = (k - _LA) % _NBUF
            gather(jc, bc).wait()
            out_copy(jc, bc).start()
        return carry

    lax.fori_loop(1, _NGRP, body, 0)

    for jc in range(_NCH - _LA, _NCH):
        bc = jc % _NBUF
        gather(jc, bc).wait()
        out_copy(jc, bc).start()
    for k in range(_NBUF):
        out_copy(_NCH - _NBUF + k, k).wait()


def kernel(y, table):
    idx = y.astype(jnp.int32).reshape(_NW, _NCH, _CH)
    out = _gather_kernel(idx, table)
    return out.reshape(BATCH, HIST, DIM_E)
